# baseline (speedup 1.0000x reference)
import functools
import math

import jax
import jax.numpy as jnp
from jax import lax
from jax.experimental import pallas as pl
from jax.experimental.pallas import tpu as pltpu
from jax.experimental.pallas import tpu_sc as plsc

D_MODEL = 128
SCALE = math.sqrt(128.0)
ROWS_PER_DMA = 128
NBUF = 5
LANES = 16


def _round_up(n, m):
    return (n + m - 1) // m * m


@functools.lru_cache(maxsize=None)
def _build(batch):
    info = plsc.get_sparse_core_info()
    nc, ns = info.num_cores, info.num_subcores
    nw = nc * ns
    per_w = batch // nw
    nchunk = per_w // ROWS_PER_DMA
    slab = _round_up(nchunk, 8) + 8
    assert per_w % ROWS_PER_DMA == 0 and nchunk % NBUF == 0

    mesh = plsc.VectorSubcoreMesh(core_axis_name="c", subcore_axis_name="s")

    @functools.partial(
        pl.kernel,
        out_type=jax.ShapeDtypeStruct((batch, D_MODEL), jnp.float32),
        mesh=mesh,
        scratch_types=[
            pltpu.VMEM((slab, ROWS_PER_DMA), jnp.int32),
            [pltpu.VMEM((ROWS_PER_DMA, D_MODEL), jnp.float32)
             for _ in range(NBUF)],
            [pltpu.SemaphoreType.DMA for _ in range(NBUF)],
            [pltpu.SemaphoreType.DMA for _ in range(NBUF)],
        ],
    )
    def emb_kernel(table_hbm, idx_hbm, out_hbm, idx_v, bufs, gsems, wsems):
        wid = lax.axis_index("s") * nc + lax.axis_index("c")
        start = wid * nchunk
        total = nw * nchunk
        a0 = pl.multiple_of(
            jnp.minimum(start - lax.rem(start, 8), total - slab), 8)
        phase = start - a0
        pltpu.sync_copy(idx_hbm.at[pl.ds(a0, slab)], idx_v)

        def gather(c, b):
            pltpu.async_copy(table_hbm.at[idx_v.at[phase + c]],
                             bufs[b], gsems[b])

        for b in range(NBUF):
            gather(b, b)

        row_base = wid * per_w

        def out_slice(c):
            return out_hbm.at[pl.ds(row_base + c * ROWS_PER_DMA,
                                    ROWS_PER_DMA)]

        @pl.loop(0, nchunk, step=NBUF)
        def _chunks(g):
            for b in range(NBUF):
                c = g + b
                buf = bufs[b]
                pltpu.make_async_copy(
                    table_hbm.at[idx_v.at[0]], buf, gsems[b]).wait()

                @plsc.parallel_loop(0, ROWS_PER_DMA, unroll=2)
                def _rows(j, buf=buf):
                    for k in range(D_MODEL // LANES):
                        sl = (j, pl.ds(k * LANES, LANES))
                        buf[sl] = buf[sl] * SCALE

                pb = (b - 1) % NBUF
                pc = c - 1 + NBUF

                @pl.when(jnp.logical_and(c >= 1, pc < nchunk))
                def _refill(pb=pb, pc=pc):
                    pltpu.make_async_copy(
                        bufs[pb], out_slice(0), wsems[pb]).wait()
                    gather(pc, pb)

                pltpu.async_copy(buf, out_slice(c), wsems[b])

        for b in range(NBUF):
            pltpu.make_async_copy(bufs[b], out_slice(0), wsems[b]).wait()

    return emb_kernel


def kernel(x, table):
    rows, cols = x.shape
    batch = rows * cols
    idx = x.astype(jnp.int32).reshape(batch // ROWS_PER_DMA, ROWS_PER_DMA)
    out = _build(batch)(table, idx)
    return out.reshape(rows, cols, D_MODEL)


# speedup vs baseline: 1.0068x; 1.0068x over previous
import functools
import math

import jax
import jax.numpy as jnp
from jax import lax
from jax.experimental import pallas as pl
from jax.experimental.pallas import tpu as pltpu
from jax.experimental.pallas import tpu_sc as plsc

D_MODEL = 128
SCALE = math.sqrt(128.0)
ROWS_PER_DMA = 64
NBUF = 10
LANES = 16


def _round_up(n, m):
    return (n + m - 1) // m * m


@functools.lru_cache(maxsize=None)
def _build(batch):
    info = plsc.get_sparse_core_info()
    nc, ns = info.num_cores, info.num_subcores
    nw = nc * ns
    per_w = batch // nw
    nchunk = per_w // ROWS_PER_DMA
    slab = _round_up(nchunk, 8) + 8
    assert per_w % ROWS_PER_DMA == 0 and nchunk % NBUF == 0

    mesh = plsc.VectorSubcoreMesh(core_axis_name="c", subcore_axis_name="s")

    @functools.partial(
        pl.kernel,
        out_type=jax.ShapeDtypeStruct((batch, D_MODEL), jnp.float32),
        mesh=mesh,
        scratch_types=[
            pltpu.VMEM((slab, ROWS_PER_DMA), jnp.int32),
            [pltpu.VMEM((ROWS_PER_DMA, D_MODEL), jnp.float32)
             for _ in range(NBUF)],
            [pltpu.SemaphoreType.DMA for _ in range(NBUF)],
            [pltpu.SemaphoreType.DMA for _ in range(NBUF)],
        ],
    )
    def emb_kernel(table_hbm, idx_hbm, out_hbm, idx_v, bufs, gsems, wsems):
        wid = lax.axis_index("s") * nc + lax.axis_index("c")
        start = wid * nchunk
        total = nw * nchunk
        a0 = pl.multiple_of(
            jnp.minimum(start - lax.rem(start, 8), total - slab), 8)
        phase = start - a0
        pltpu.sync_copy(idx_hbm.at[pl.ds(a0, slab)], idx_v)

        def gather(c, b):
            pltpu.async_copy(table_hbm.at[idx_v.at[phase + c]],
                             bufs[b], gsems[b])

        for b in range(NBUF):
            gather(b, b)

        row_base = wid * per_w

        def out_slice(c):
            return out_hbm.at[pl.ds(row_base + c * ROWS_PER_DMA,
                                    ROWS_PER_DMA)]

        @pl.loop(0, nchunk, step=NBUF)
        def _chunks(g):
            for b in range(NBUF):
                c = g + b
                buf = bufs[b]
                pltpu.make_async_copy(
                    table_hbm.at[idx_v.at[0]], buf, gsems[b]).wait()

                @plsc.parallel_loop(0, ROWS_PER_DMA, unroll=2)
                def _rows(j, buf=buf):
                    for k in range(D_MODEL // LANES):
                        sl = (j, pl.ds(k * LANES, LANES))
                        buf[sl] = buf[sl] * SCALE

                pb = (b - 1) % NBUF
                pc = c - 1 + NBUF

                @pl.when(jnp.logical_and(c >= 1, pc < nchunk))
                def _refill(pb=pb, pc=pc):
                    pltpu.make_async_copy(
                        bufs[pb], out_slice(0), wsems[pb]).wait()
                    gather(pc, pb)

                pltpu.async_copy(buf, out_slice(c), wsems[b])

        for b in range(NBUF):
            pltpu.make_async_copy(bufs[b], out_slice(0), wsems[b]).wait()

    return emb_kernel


def kernel(x, table):
    rows, cols = x.shape
    batch = rows * cols
    idx = x.astype(jnp.int32).reshape(batch // ROWS_PER_DMA, ROWS_PER_DMA)
    out = _build(batch)(table, idx)
    return out.reshape(rows, cols, D_MODEL)
